# baseline (speedup 1.0000x reference)
import functools

import jax
import jax.numpy as jnp
from jax import lax
from jax.experimental import pallas as pl
from jax.experimental.pallas import tpu as pltpu
from jax.experimental.pallas import tpu_sc as plsc

B = 16384
V = 100000
D = 32
NU = 8
NI = 4
TW = 8192
NT = 13
VP = NT * TW // 8

NC = 2
NS = 16
NW = NC * NS

CH = 128
GRP = 8
NCH = B // CH
U_WPF = NW // NU
I_WPF = NW // NI
U_CPW = NCH // U_WPF
I_CPW = NCH // I_WPF

BM = 2048


def _rnd_bf16_bits(x):
    fb = lax.bitcast_convert_type(x, jnp.int32)
    r = lax.shift_right_logical(fb, 16) & 1
    return lax.shift_right_logical(fb + 0x7FFF + r, 16)


def _pack_body(x_ref, out_ref):
    x = x_ref[0]
    q = TW // 8
    xs_lo = jnp.concatenate([x[:D // 2, j * q:(j + 1) * q] for j in range(8)],
                            axis=0)
    xs_hi = jnp.concatenate([x[D // 2:, j * q:(j + 1) * q] for j in range(8)],
                            axis=0)
    w = (_rnd_bf16_bits(xs_lo) & 0xFFFF) | (_rnd_bf16_bits(xs_hi) << 16)
    out_ref[...] = w.T


def _pack_table(tabT):
    F = tabT.shape[0]
    return pl.pallas_call(
        _pack_body,
        grid=(F, NT),
        in_specs=[pl.BlockSpec((1, D, TW), lambda f, t: (f, 0, t))],
        out_specs=pl.BlockSpec((TW // 8, 4 * D), lambda f, t: (f * NT + t, 0)),
        out_shape=jax.ShapeDtypeStruct((F * VP, 4 * D), jnp.int32),
    )(tabT)


SLOTS = 4


def _sc_gather_body(wpf, cpw, tab, idxT, offT, xT,
                    idx_v, off_v, rows_v, out_v, sem):
    wid = lax.axis_index("s") * NC + lax.axis_index("c")
    iota = lax.iota(jnp.int32, 16)

    def tower(tab, idxT, offT, out_hbm, wpf, cpw):
        f = wid // wpf
        cb0 = (wid % wpf) * cpw
        pltpu.sync_copy(idxT.at[f, pl.ds(cb0, cpw)], idx_v.at[pl.ds(0, cpw)])
        pltpu.sync_copy(offT.at[f, pl.ds(cb0, cpw)], off_v.at[pl.ds(0, cpw)])

        def group(g, _):
            cb = cb0 + g * SLOTS
            cps = []
            for s in range(SLOTS):
                cps.append(pltpu.async_copy(tab.at[idx_v.at[g * SLOTS + s]],
                                            rows_v.at[s], sem))
            for s in range(SLOTS):
                cps[s].wait()

                def extract(r0, _):
                    off16 = off_v[g * SLOTS + s, pl.ds(r0 * 16, 16)]
                    rows16 = r0 * 16 + iota
                    for k in range(16):
                        evec = (iota + k) & 15
                        w = plsc.load_gather(rows_v.at[s],
                                             [rows16, off16 + evec])
                        lo = plsc.bitcast(w << 16, jnp.float32)
                        hi = plsc.bitcast(w & jnp.int32(-65536), jnp.float32)
                        plsc.store_scatter(out_v.at[s], [evec, rows16], lo)
                        plsc.store_scatter(out_v.at[s], [evec + D // 2,
                                                         rows16], hi)
                    return _

                lax.fori_loop(0, CH // 16, extract, 0)
                pltpu.sync_copy(
                    out_v.at[s],
                    out_hbm.at[pl.ds(f * D, D), pl.ds((cb + s) * CH, CH)])
            return _

        lax.fori_loop(0, cpw // SLOTS, group, 0)

    tower(tab, idxT, offT, xT, wpf, cpw)


def _gather_tower(tab, idxT, offT, nf, wpf, cpw):
    mesh = plsc.VectorSubcoreMesh(core_axis_name="c", subcore_axis_name="s")
    f = functools.partial(
        pl.kernel,
        out_type=jax.ShapeDtypeStruct((nf * D, B), jnp.float32),
        mesh=mesh,
        scratch_types=[pltpu.VMEM((cpw, CH), jnp.int32),
                       pltpu.VMEM((cpw, CH), jnp.int32),
                       pltpu.VMEM((SLOTS, CH, 4 * D), jnp.int32),
                       pltpu.VMEM((SLOTS, D, CH), jnp.float32),
                       pltpu.SemaphoreType.DMA],
        compiler_params=pltpu.CompilerParams(needs_layout_passes=False),
    )(functools.partial(_sc_gather_body, wpf, cpw))
    return f(tab, idxT, offT)


def _tower_body(xt_ref, w1_ref, b1_ref, g1_ref, be1_ref, w2_ref, b2_ref,
                z2_ref, st2_ref, z1_scr, st1_scr):
    g = pl.program_id(0)
    nb = pl.num_programs(0) // 2

    @pl.when(g == 0)
    def _():
        st1_scr[...] = jnp.zeros_like(st1_scr)

    @pl.when(g < nb)
    def _():
        z = lax.dot_general(xt_ref[...], w1_ref[...],
                            (((0,), (0,)), ((), ())),
                            preferred_element_type=jnp.float32) + b1_ref[...]
        i = g
        z1_scr[pl.ds(i * BM, BM), :] = z
        st1_scr[...] += jnp.concatenate(
            [jnp.sum(z, axis=0, keepdims=True),
             jnp.sum(z * z, axis=0, keepdims=True)], axis=0)

    @pl.when(g == nb)
    def _():
        st2_ref[...] = jnp.zeros_like(st2_ref)

    @pl.when(g >= nb)
    def _():
        i = g - nb
        z1 = z1_scr[pl.ds(i * BM, BM), :]
        h = jnp.maximum(_bn(z1, st1_scr[...], g1_ref[...], be1_ref[...]), 0.0)
        z2 = jnp.dot(h, w2_ref[...],
                     preferred_element_type=jnp.float32) + b2_ref[...]
        z2_ref[...] = z2.astype(jnp.bfloat16)
        st2_ref[...] += jnp.concatenate(
            [jnp.sum(z2, axis=0, keepdims=True),
             jnp.sum(z2 * z2, axis=0, keepdims=True)], axis=0)


def _tower(xt, w1, b1, g1, be1, w2, b2):
    K, Bn = xt.shape
    N1 = w1.shape[1]
    N2 = w2.shape[1]
    nb = Bn // BM
    cst = lambda i, j=0: (lambda g: (i, j))
    return pl.pallas_call(
        _tower_body,
        grid=(2 * nb,),
        in_specs=[pl.BlockSpec((K, BM),
                               lambda g: (0, jnp.where(g < nb, g, nb - 1))),
                  pl.BlockSpec((K, N1), cst(0)),
                  pl.BlockSpec((1, N1), cst(0)),
                  pl.BlockSpec((1, N1), cst(0)),
                  pl.BlockSpec((1, N1), cst(0)),
                  pl.BlockSpec((N1, N2), cst(0)),
                  pl.BlockSpec((1, N2), cst(0))],
        out_specs=[pl.BlockSpec((BM, N2),
                                lambda g: (jnp.where(g < nb, 0, g - nb), 0)),
                   pl.BlockSpec((2, N2), cst(0))],
        out_shape=[jax.ShapeDtypeStruct((Bn, N2), jnp.bfloat16),
                   jax.ShapeDtypeStruct((2, N2), jnp.float32)],
        scratch_shapes=[pltpu.VMEM((Bn, N1), jnp.float32),
                        pltpu.VMEM((2, N1), jnp.float32)],
    )(xt, w1, b1, g1, be1, w2, b2)


def _bn(z, st, g, be):
    mu = st[0:1, :] * (1.0 / B)
    var = st[1:2, :] * (1.0 / B) - mu * mu
    return (z - mu) / jnp.sqrt(var + 1e-5) * g + be


def _final_body(zu_ref, stu_ref, gu_ref, beu_ref,
                zi_ref, sti_ref, gi_ref, bei_ref, sw_ref, y_ref):
    u = jnp.maximum(_bn(zu_ref[...].astype(jnp.float32), stu_ref[...],
                        gu_ref[...], beu_ref[...]), 0.0)
    v = jnp.maximum(_bn(zi_ref[...].astype(jnp.float32), sti_ref[...],
                        gi_ref[...], bei_ref[...]), 0.0)
    dot = jnp.sum(u * v, axis=1, keepdims=True)
    un = jnp.sqrt(jnp.sum(u * u, axis=1, keepdims=True))
    vn = jnp.sqrt(jnp.sum(v * v, axis=1, keepdims=True))
    y_ref[...] = dot / jnp.maximum(un * vn, 1e-8) - jnp.log(sw_ref[...])


def _final(zu, stu, gu, beu, zi, sti, gi, bei, sw):
    Ku = zu.shape[1]
    Ki = zi.shape[1]
    return pl.pallas_call(
        _final_body,
        grid=(B // BM,),
        in_specs=[pl.BlockSpec((BM, Ku), lambda i: (i, 0)),
                  pl.BlockSpec((2, Ku), lambda i: (0, 0)),
                  pl.BlockSpec((1, Ku), lambda i: (0, 0)),
                  pl.BlockSpec((1, Ku), lambda i: (0, 0)),
                  pl.BlockSpec((BM, Ki), lambda i: (i, 0)),
                  pl.BlockSpec((2, Ki), lambda i: (0, 0)),
                  pl.BlockSpec((1, Ki), lambda i: (0, 0)),
                  pl.BlockSpec((1, Ki), lambda i: (0, 0)),
                  pl.BlockSpec((BM, 1), lambda i: (i, 0))],
        out_specs=pl.BlockSpec((BM, 1), lambda i: (i, 0)),
        out_shape=jax.ShapeDtypeStruct((B, 1), jnp.float32),
    )(zu, stu, gu, beu, zi, sti, gi, bei, sw)


def _scores_body(y_ref, out_ref):
    y = y_ref[...]
    cols = [y]
    for j in range(1, 4):
        cols.append(jnp.concatenate([y[j:, :], y[:j, :]], axis=0))
    out_ref[...] = jnp.concatenate(cols, axis=1)


def _scores(y_col):
    return pl.pallas_call(
        _scores_body,
        out_shape=jax.ShapeDtypeStruct((B, 4), jnp.float32),
    )(y_col)


def kernel(user_ids, item_ids, sample_weight, user_tables, item_tables,
           uW1, ub1, ug1, ube1, uW2, ub2, ug2, ube2,
           iW1, ib1, ig1, ibe1, iW2, ib2, ig2, ibe2):
    upk = _pack_table(jnp.transpose(user_tables, (0, 2, 1)))
    ipk = _pack_table(jnp.transpose(item_tables, (0, 2, 1)))

    uid = user_ids.astype(jnp.int32)
    iid = item_ids.astype(jnp.int32)
    urow = (((uid >> 13) << 10) + (uid & 1023)
            + (jnp.arange(NU, dtype=jnp.int32) * VP)[None, :])
    irow = (((iid >> 13) << 10) + (iid & 1023)
            + (jnp.arange(NI, dtype=jnp.int32) * VP)[None, :])
    uoff = ((uid >> 10) & 7) * (D // 2)
    ioff = ((iid >> 10) & 7) * (D // 2)

    uidxT = urow.T.reshape(NU, NCH, CH)
    iidxT = irow.T.reshape(NI, NCH, CH)
    uoffT = uoff.T.reshape(NU, NCH, CH)
    ioffT = ioff.T.reshape(NI, NCH, CH)

    xuT = _gather_tower(upk, uidxT, uoffT, NU, U_WPF, U_CPW)
    xiT = _gather_tower(ipk, iidxT, ioffT, NI, I_WPF, I_CPW)

    z2u, s2u = _tower(xuT, uW1, ub1.reshape(1, -1), ug1.reshape(1, -1),
                      ube1.reshape(1, -1), uW2, ub2.reshape(1, -1))
    z2i, s2i = _tower(xiT, iW1, ib1.reshape(1, -1), ig1.reshape(1, -1),
                      ibe1.reshape(1, -1), iW2, ib2.reshape(1, -1))

    y = _final(z2u, s2u, ug2.reshape(1, -1), ube2.reshape(1, -1),
               z2i, s2i, ig2.reshape(1, -1), ibe2.reshape(1, -1),
               sample_weight.reshape(B, 1))

    return _scores(y)


# speedup vs baseline: 1.0512x; 1.0512x over previous
import functools

import jax
import jax.numpy as jnp
from jax import lax
from jax.experimental import pallas as pl
from jax.experimental.pallas import tpu as pltpu
from jax.experimental.pallas import tpu_sc as plsc

B = 16384
V = 100000
D = 32
NU = 8
NI = 4
TW = 8192
NT = 13
VP = NT * TW // 8

NC = 2
NS = 16
NW = NC * NS

CH = 128
NCH = B // CH
U_WPF = NW // NU
I_WPF = NW // NI
U_CPW = NCH // U_WPF
I_CPW = NCH // I_WPF

BM = 2048


def _rnd_bf16_bits(x):
    fb = lax.bitcast_convert_type(x, jnp.int32)
    r = lax.shift_right_logical(fb, 16) & 1
    return lax.shift_right_logical(fb + 0x7FFF + r, 16)


def _pack_body(x_ref, out_ref):
    x = x_ref[0]
    q = TW // 8
    xs_lo = jnp.concatenate([x[:D // 2, j * q:(j + 1) * q] for j in range(8)],
                            axis=0)
    xs_hi = jnp.concatenate([x[D // 2:, j * q:(j + 1) * q] for j in range(8)],
                            axis=0)
    w = (_rnd_bf16_bits(xs_lo) & 0xFFFF) | (_rnd_bf16_bits(xs_hi) << 16)
    out_ref[...] = w.T


def _pack_table(tabT):
    F = tabT.shape[0]
    return pl.pallas_call(
        _pack_body,
        grid=(F, NT),
        in_specs=[pl.BlockSpec((1, D, TW), lambda f, t: (f, 0, t))],
        out_specs=pl.BlockSpec((TW // 8, 4 * D), lambda f, t: (f * NT + t, 0)),
        out_shape=jax.ShapeDtypeStruct((F * VP, 4 * D), jnp.int32),
    )(tabT)


SLOTS = 4


def _sc_gather_body(wpf, cpw, tab, idxT, offT, xT,
                    idx_v, off_v, rows_v, out_v, sem):
    wid = lax.axis_index("s") * NC + lax.axis_index("c")
    iota = lax.iota(jnp.int32, 16)

    def tower(tab, idxT, offT, out_hbm, wpf, cpw):
        f = wid // wpf
        cb0 = (wid % wpf) * cpw
        pltpu.sync_copy(idxT.at[f, pl.ds(cb0, cpw)], idx_v.at[pl.ds(0, cpw)])
        pltpu.sync_copy(offT.at[f, pl.ds(cb0, cpw)], off_v.at[pl.ds(0, cpw)])

        def group(g, _):
            cb = cb0 + g * SLOTS
            cps = []
            for s in range(SLOTS):
                cps.append(pltpu.async_copy(tab.at[idx_v.at[g * SLOTS + s]],
                                            rows_v.at[s], sem))
            for s in range(SLOTS):
                cps[s].wait()

                def extract(r0, _):
                    off16 = off_v[g * SLOTS + s, pl.ds(r0 * 16, 16)]
                    rows16 = r0 * 16 + iota
                    for k in range(16):
                        evec = (iota + k) & 15
                        w = plsc.load_gather(rows_v.at[s],
                                             [rows16, off16 + evec])
                        lo = plsc.bitcast(w << 16, jnp.float32)
                        hi = plsc.bitcast(w & jnp.int32(-65536), jnp.float32)
                        plsc.store_scatter(out_v.at[s], [evec, rows16], lo)
                        plsc.store_scatter(out_v.at[s], [evec + D // 2,
                                                         rows16], hi)
                    return _

                lax.fori_loop(0, CH // 16, extract, 0)
                pltpu.sync_copy(
                    out_v.at[s],
                    out_hbm.at[pl.ds(f * D, D), pl.ds((cb + s) * CH, CH)])
            return _

        lax.fori_loop(0, cpw // SLOTS, group, 0)

    tower(tab, idxT, offT, xT, wpf, cpw)


def _gather_tower(tab, idxT, offT, nf, wpf, cpw):
    mesh = plsc.VectorSubcoreMesh(core_axis_name="c", subcore_axis_name="s")
    f = functools.partial(
        pl.kernel,
        out_type=jax.ShapeDtypeStruct((nf * D, B), jnp.float32),
        mesh=mesh,
        scratch_types=[pltpu.VMEM((cpw, CH), jnp.int32),
                       pltpu.VMEM((cpw, CH), jnp.int32),
                       pltpu.VMEM((SLOTS, CH, 4 * D), jnp.int32),
                       pltpu.VMEM((SLOTS, D, CH), jnp.float32),
                       pltpu.SemaphoreType.DMA],
        compiler_params=pltpu.CompilerParams(needs_layout_passes=False),
    )(functools.partial(_sc_gather_body, wpf, cpw))
    return f(tab, idxT, offT)


def _tower_body(xt_ref, w1_ref, b1_ref, g1_ref, be1_ref, w2_ref, b2_ref,
                z2_ref, st2_ref, z1_scr, st1_scr):
    g = pl.program_id(0)
    nb = pl.num_programs(0) // 2

    @pl.when(g == 0)
    def _():
        st1_scr[...] = jnp.zeros_like(st1_scr)

    @pl.when(g < nb)
    def _():
        z = lax.dot_general(xt_ref[...], w1_ref[...],
                            (((0,), (0,)), ((), ())),
                            preferred_element_type=jnp.float32) + b1_ref[...]
        i = g
        z1_scr[pl.ds(i * BM, BM), :] = z
        st1_scr[...] += jnp.concatenate(
            [jnp.sum(z, axis=0, keepdims=True),
             jnp.sum(z * z, axis=0, keepdims=True)], axis=0)

    @pl.when(g == nb)
    def _():
        st2_ref[...] = jnp.zeros_like(st2_ref)

    @pl.when(g >= nb)
    def _():
        i = g - nb
        z1 = z1_scr[pl.ds(i * BM, BM), :]
        h = jnp.maximum(_bn(z1, st1_scr[...], g1_ref[...], be1_ref[...]), 0.0)
        z2 = jnp.dot(h, w2_ref[...],
                     preferred_element_type=jnp.float32) + b2_ref[...]
        z2_ref[...] = z2.astype(jnp.bfloat16)
        st2_ref[...] += jnp.concatenate(
            [jnp.sum(z2, axis=0, keepdims=True),
             jnp.sum(z2 * z2, axis=0, keepdims=True)], axis=0)


def _tower(xt, w1, b1, g1, be1, w2, b2):
    K, Bn = xt.shape
    N1 = w1.shape[1]
    N2 = w2.shape[1]
    nb = Bn // BM
    cst = lambda i, j=0: (lambda g: (i, j))
    return pl.pallas_call(
        _tower_body,
        grid=(2 * nb,),
        in_specs=[pl.BlockSpec((K, BM),
                               lambda g: (0, jnp.where(g < nb, g, nb - 1))),
                  pl.BlockSpec((K, N1), cst(0)),
                  pl.BlockSpec((1, N1), cst(0)),
                  pl.BlockSpec((1, N1), cst(0)),
                  pl.BlockSpec((1, N1), cst(0)),
                  pl.BlockSpec((N1, N2), cst(0)),
                  pl.BlockSpec((1, N2), cst(0))],
        out_specs=[pl.BlockSpec((BM, N2),
                                lambda g: (jnp.where(g < nb, 0, g - nb), 0)),
                   pl.BlockSpec((2, N2), cst(0))],
        out_shape=[jax.ShapeDtypeStruct((Bn, N2), jnp.bfloat16),
                   jax.ShapeDtypeStruct((2, N2), jnp.float32)],
        scratch_shapes=[pltpu.VMEM((Bn, N1), jnp.float32),
                        pltpu.VMEM((2, N1), jnp.float32)],
    )(xt, w1, b1, g1, be1, w2, b2)


def _bn(z, st, g, be):
    mu = st[0:1, :] * (1.0 / B)
    var = st[1:2, :] * (1.0 / B) - mu * mu
    return (z - mu) / jnp.sqrt(var + 1e-5) * g + be


def _final_body(zu_ref, stu_ref, gu_ref, beu_ref,
                zi_ref, sti_ref, gi_ref, bei_ref, sw_ref, y_ref):
    u = jnp.maximum(_bn(zu_ref[...].astype(jnp.float32), stu_ref[...],
                        gu_ref[...], beu_ref[...]), 0.0)
    v = jnp.maximum(_bn(zi_ref[...].astype(jnp.float32), sti_ref[...],
                        gi_ref[...], bei_ref[...]), 0.0)
    dot = jnp.sum(u * v, axis=1, keepdims=True)
    un = jnp.sqrt(jnp.sum(u * u, axis=1, keepdims=True))
    vn = jnp.sqrt(jnp.sum(v * v, axis=1, keepdims=True))
    y_ref[...] = dot / jnp.maximum(un * vn, 1e-8) - jnp.log(sw_ref[...])


def _final(zu, stu, gu, beu, zi, sti, gi, bei, sw):
    Ku = zu.shape[1]
    Ki = zi.shape[1]
    return pl.pallas_call(
        _final_body,
        grid=(B // BM,),
        in_specs=[pl.BlockSpec((BM, Ku), lambda i: (i, 0)),
                  pl.BlockSpec((2, Ku), lambda i: (0, 0)),
                  pl.BlockSpec((1, Ku), lambda i: (0, 0)),
                  pl.BlockSpec((1, Ku), lambda i: (0, 0)),
                  pl.BlockSpec((BM, Ki), lambda i: (i, 0)),
                  pl.BlockSpec((2, Ki), lambda i: (0, 0)),
                  pl.BlockSpec((1, Ki), lambda i: (0, 0)),
                  pl.BlockSpec((1, Ki), lambda i: (0, 0)),
                  pl.BlockSpec((BM, 1), lambda i: (i, 0))],
        out_specs=pl.BlockSpec((BM, 1), lambda i: (i, 0)),
        out_shape=jax.ShapeDtypeStruct((B, 1), jnp.float32),
    )(zu, stu, gu, beu, zi, sti, gi, bei, sw)


def _scores_body(y_ref, out_ref):
    y = y_ref[...]
    rows = [y]
    for j in range(1, 4):
        rows.append(jnp.concatenate([y[:, j:], y[:, :j]], axis=1))
    out_ref[...] = jnp.concatenate(rows, axis=0)


def _scores(y_row):
    return pl.pallas_call(
        _scores_body,
        out_shape=jax.ShapeDtypeStruct((4, B), jnp.float32),
    )(y_row)


def kernel(user_ids, item_ids, sample_weight, user_tables, item_tables,
           uW1, ub1, ug1, ube1, uW2, ub2, ug2, ube2,
           iW1, ib1, ig1, ibe1, iW2, ib2, ig2, ibe2):
    upk = _pack_table(jnp.transpose(user_tables, (0, 2, 1)))
    ipk = _pack_table(jnp.transpose(item_tables, (0, 2, 1)))

    uid = user_ids.astype(jnp.int32)
    iid = item_ids.astype(jnp.int32)
    urow = (((uid >> 13) << 10) + (uid & 1023)
            + (jnp.arange(NU, dtype=jnp.int32) * VP)[None, :])
    irow = (((iid >> 13) << 10) + (iid & 1023)
            + (jnp.arange(NI, dtype=jnp.int32) * VP)[None, :])
    uoff = ((uid >> 10) & 7) * (D // 2)
    ioff = ((iid >> 10) & 7) * (D // 2)

    uidxT = urow.T.reshape(NU, NCH, CH)
    iidxT = irow.T.reshape(NI, NCH, CH)
    uoffT = uoff.T.reshape(NU, NCH, CH)
    ioffT = ioff.T.reshape(NI, NCH, CH)

    xuT = _gather_tower(upk, uidxT, uoffT, NU, U_WPF, U_CPW)
    xiT = _gather_tower(ipk, iidxT, ioffT, NI, I_WPF, I_CPW)

    z2u, s2u = _tower(xuT, uW1, ub1.reshape(1, -1), ug1.reshape(1, -1),
                      ube1.reshape(1, -1), uW2, ub2.reshape(1, -1))
    z2i, s2i = _tower(xiT, iW1, ib1.reshape(1, -1), ig1.reshape(1, -1),
                      ibe1.reshape(1, -1), iW2, ib2.reshape(1, -1))

    y = _final(z2u, s2u, ug2.reshape(1, -1), ube2.reshape(1, -1),
               z2i, s2i, ig2.reshape(1, -1), ibe2.reshape(1, -1),
               sample_weight.reshape(B, 1))

    sc4 = _scores(y.reshape(1, B))
    return sc4.T
